# baseline (speedup 1.0000x reference)
import jax
import jax.numpy as jnp
from jax import lax
from jax.experimental import pallas as pl
from jax.experimental.pallas import tpu as pltpu
from jax.experimental.pallas import tpu_sc as plsc

_NC = 2
_NS = 16
_NW = _NC * _NS
_LANES = 16

_B = 4
_S = 4096
_D = 1024
_CR = 8
_CHUNK = _B * _CR * _D
_SEQ_PER_W = _S // _NW
_NCHUNK = _SEQ_PER_W // _CR
_XBUFS = 3
_PEBUFS = 3


def _sc_body(x_hbm, pe_hbm, o_hbm, xbufs, pebufs, sin, sout, spe):
    wid = lax.axis_index("s") * _NC + lax.axis_index("c")
    row0 = wid * _SEQ_PER_W

    def start_in(c):
        r = row0 + c * _CR
        return pltpu.async_copy(
            x_hbm.at[:, pl.ds(r, _CR), :], xbufs[c % _XBUFS], sin[c % _XBUFS]
        )

    def start_pe(c):
        return pltpu.async_copy(
            pe_hbm.at[pl.ds(row0 + c * _CR, _CR), :],
            pebufs[c % _PEBUFS],
            spe[c % _PEBUFS],
        )

    pre = _XBUFS - 1
    in_h = {}
    out_h = {}
    pe_h = {}
    for c in range(min(pre, _NCHUNK)):
        in_h[c] = start_in(c)
        pe_h[c] = start_pe(c)

    for c in range(_NCHUNK):
        sl = c % _XBUFS
        xbuf = xbufs[sl]
        pebuf = pebufs[c % _PEBUFS]

        nc = c + pre
        if nc < _NCHUNK:
            if nc - _XBUFS >= 0:
                out_h.pop(nc - _XBUFS).wait()
            in_h[nc] = start_in(nc)
            pe_h[nc] = start_pe(nc)

        pe_h.pop(c).wait()
        in_h.pop(c).wait()

        @plsc.parallel_loop(0, _CR * _D, step=_LANES, unroll=4)
        def add_loop(o):
            r = lax.shift_right_logical(o, 10)
            col = pl.multiple_of(lax.bitwise_and(o, _D - 1), _LANES)
            pv = pebuf[r, pl.ds(col, _LANES)]
            for b in range(_B):
                xbuf[b, r, pl.ds(col, _LANES)] = xbuf[b, r, pl.ds(col, _LANES)] + pv

        r = row0 + c * _CR
        out_h[c] = pltpu.async_copy(
            xbuf, o_hbm.at[:, pl.ds(r, _CR), :], sout[sl]
        )

    for c in sorted(out_h):
        out_h.pop(c).wait()


def kernel(x, pe_table):
    B, S, D = x.shape

    sc_call = pl.kernel(
        _sc_body,
        out_type=jax.ShapeDtypeStruct((B, S, D), x.dtype),
        mesh=plsc.VectorSubcoreMesh(core_axis_name="c", subcore_axis_name="s"),
        compiler_params=pltpu.CompilerParams(use_tc_tiling_on_sc=True),
        scratch_types=[
            [pltpu.VMEM((_B, _CR, _D), jnp.float32) for _ in range(_XBUFS)],
            [pltpu.VMEM((_CR, _D), jnp.float32) for _ in range(_PEBUFS)],
            [pltpu.SemaphoreType.DMA for _ in range(_XBUFS)],
            [pltpu.SemaphoreType.DMA for _ in range(_XBUFS)],
            [pltpu.SemaphoreType.DMA for _ in range(_PEBUFS)],
        ],
    )
    return sc_call(x, pe_table)
